# baseline (speedup 1.0000x reference)
import functools

import jax
import jax.numpy as jnp
from jax import lax
from jax.experimental import pallas as pl
from jax.experimental.pallas import tpu as pltpu
from jax.experimental.pallas import tpu_sc as plsc

N_NODES = 10000
N_EDGES = 320000
D = 128

NC = 2
NS = 16
NW = NC * NS

CHUNK = 128
N_CHUNKS = N_EDGES // CHUNK
NBUF = 3
CPT = N_CHUNKS // NW
N_EXTRA = N_CHUNKS - CPT * NW
T_GROUPS = CPT // NBUF
IDX_CLAMP = N_CHUNKS - NBUF
RB = 40
N_RBLOCKS = N_NODES // RB

_mesh = plsc.VectorSubcoreMesh(core_axis_name="c", subcore_axis_name="s")


@functools.partial(
    pl.kernel,
    out_type=jax.ShapeDtypeStruct((NC * N_NODES, D), jnp.float32),
    mesh=_mesh,
    scratch_types=[
        [pltpu.VMEM((NBUF, 1, CHUNK), jnp.int32) for _ in range(2)],
        [pltpu.VMEM((NBUF, 1, CHUNK), jnp.int32) for _ in range(2)],
        [pltpu.VMEM((CHUNK, D), jnp.float32) for _ in range(NBUF)],
        pltpu.VMEM_SHARED((N_NODES, D), jnp.float32),
        [pltpu.SemaphoreType.DMA for _ in range(NBUF)],
        [pltpu.SemaphoreType.DMA for _ in range(NBUF)],
        [pltpu.SemaphoreType.DMA for _ in range(2)],
        pltpu.SemaphoreType.DMA,
    ],
)
def _sc_aggregate(x_hbm, src_hbm, dst_hbm, out_hbm,
                  sblk, dblk, rows, acc_sh, sg, ss, si, sz):
    c = lax.axis_index("c")
    s = lax.axis_index("s")
    wid = s * NC + c

    zeros16 = jnp.zeros((16,), jnp.float32)

    def zstore(i, carry):
        rows[0][i // 8, pl.ds((i % 8) * 16, 16)] = zeros16
        return carry

    lax.fori_loop(0, RB * 8, zstore, 0)

    nrb = N_RBLOCKS // NS + jnp.where(s < N_RBLOCKS % NS, 1, 0)
    zsrc = rows[0].at[pl.ds(0, RB)]

    def zfire(i, carry):
        pltpu.async_copy(zsrc, acc_sh.at[pl.ds((s + i * NS) * RB, RB)], sz)
        return carry

    def zdrain(i, carry):
        pltpu.make_async_copy(zsrc, acc_sh.at[pl.ds(0, RB)], sz).wait()
        return carry

    lax.fori_loop(0, nrb, zfire, 0)

    cstart = CPT * wid + jnp.minimum(wid, N_EXTRA)

    def idxload(p, g):
        row = jnp.minimum(cstart + NBUF * g, IDX_CLAMP)
        pltpu.async_copy(src_hbm.at[pl.ds(row, NBUF)], sblk[p], si[p])
        pltpu.async_copy(dst_hbm.at[pl.ds(row, NBUF)], dblk[p], si[p])

    def idxwait(p):
        pltpu.make_async_copy(src_hbm.at[pl.ds(0, NBUF)], sblk[p], si[p]).wait()
        pltpu.make_async_copy(dst_hbm.at[pl.ds(0, NBUF)], dblk[p], si[p]).wait()

    def fire_gather(p, b):
        pltpu.async_copy(x_hbm.at[sblk[p].at[b].at[0]], rows[b], sg[b])

    def wait_gather(p, b):
        pltpu.make_async_copy(x_hbm.at[sblk[p].at[b].at[0]], rows[b], sg[b]).wait()

    def fire_scatter(p, b):
        pltpu.async_copy(rows[b], acc_sh.at[dblk[p].at[b].at[0]], ss[b], add=True)

    def wait_scatter(p, b):
        pltpu.make_async_copy(rows[b], acc_sh.at[dblk[p].at[b].at[0]], ss[b]).wait()

    idxload(0, 0)
    idxwait(0)
    fire_gather(0, 1)
    fire_gather(0, 2)
    lax.fori_loop(0, nrb, zdrain, 0)
    fire_gather(0, 0)
    idxload(1, 1)

    plsc.subcore_barrier()

    def body(u, carry):
        for b in range(NBUF):
            wait_gather(0, b)
            fire_scatter(0, b)
        idxwait(1)
        for b in range(NBUF):
            wait_scatter(0, b)
            fire_gather(1, b)
        idxload(0, 2 * u + 2)
        for b in range(NBUF):
            wait_gather(1, b)
            fire_scatter(1, b)
        idxwait(0)
        for b in range(NBUF):
            wait_scatter(1, b)
            fire_gather(0, b)
        idxload(1, 2 * u + 3)
        return carry

    lax.fori_loop(0, T_GROUPS // 2, body, 0)

    for b in range(NBUF):
        wait_gather(0, b)
    idxwait(1)

    @pl.when(wid < N_EXTRA)
    def _():
        fire_scatter(0, 0)
        wait_scatter(0, 0)

    plsc.subcore_barrier()

    def ffire(i, carry):
        r0 = (s + i * NS) * RB
        pltpu.async_copy(
            acc_sh.at[pl.ds(r0, RB)],
            out_hbm.at[pl.ds(c * N_NODES + r0, RB)],
            sz,
        )
        return carry

    def fdrain(i, carry):
        pltpu.make_async_copy(
            acc_sh.at[pl.ds(0, RB)], out_hbm.at[pl.ds(0, RB)], sz,
        ).wait()
        return carry

    lax.fori_loop(0, nrb, ffire, 0)
    lax.fori_loop(0, nrb, fdrain, 0)


def _mm_body(p0_ref, p1_ref, w_ref, o_ref):
    o_ref[...] = jnp.dot(
        p0_ref[...] + p1_ref[...], w_ref[...],
        preferred_element_type=jnp.float32,
    )


_BM = 2000
_NBLK = N_NODES // _BM


def _tc_matmul(partials, W):
    return pl.pallas_call(
        _mm_body,
        grid=(_NBLK,),
        in_specs=[
            pl.BlockSpec((_BM, D), lambda i: (i, 0)),
            pl.BlockSpec((_BM, D), lambda i: (i + _NBLK, 0)),
            pl.BlockSpec((D, D), lambda i: (0, 0)),
        ],
        out_specs=pl.BlockSpec((_BM, D), lambda i: (i, 0)),
        out_shape=jax.ShapeDtypeStruct((N_NODES, D), jnp.float32),
    )(partials, partials, W)


@jax.jit
def kernel(x, edge_index, W):
    src3 = edge_index[0].astype(jnp.int32).reshape(N_CHUNKS, 1, CHUNK)
    dst3 = edge_index[1].astype(jnp.int32).reshape(N_CHUNKS, 1, CHUNK)
    partials = _sc_aggregate(x, src3, dst3)
    return _tc_matmul(partials, W)
